# Initial kernel scaffold; baseline (speedup 1.0000x reference)
import jax
import jax.numpy as jnp
from jax.experimental import pallas as pl


def kernel(x_cat, x_cont, cat_tables, cat_W, cat_b, cont_W, cont_b, cls_token):
    raise NotImplementedError("write your pallas kernel here")


# speedup vs baseline: 6.6335x; 6.6335x over previous
import functools

import jax
import jax.numpy as jnp
from jax import lax
from jax.experimental import pallas as pl
from jax.experimental.pallas import tpu as pltpu
from jax.experimental.pallas import tpu_sc as plsc

_NC = 2
_NS = 16
_NW = _NC * _NS
_CHUNK = 128


def _sc_gather(table2d, idx3d, n_rows, emb_dim):
    chunks = idx3d.shape[1]
    rows_per_worker = chunks * _CHUNK
    mesh = plsc.VectorSubcoreMesh(core_axis_name="c", subcore_axis_name="s")

    @functools.partial(
        pl.kernel,
        out_type=jax.ShapeDtypeStruct((n_rows, emb_dim), jnp.float32),
        mesh=mesh,
        scratch_types=[
            pltpu.VMEM((chunks, _CHUNK), jnp.int32),
            pltpu.VMEM((_CHUNK, emb_dim), jnp.float32),
            pltpu.SemaphoreType.DMA,
        ],
        compiler_params=pltpu.CompilerParams(use_tc_tiling_on_sc=False),
    )
    def gather_kernel(table_hbm, idx_hbm, out_hbm, idx_v, rows_v, sem):
        wid = lax.axis_index("s") * _NC + lax.axis_index("c")
        pltpu.sync_copy(idx_hbm.at[wid], idx_v)
        base = pl.multiple_of(wid * rows_per_worker, _CHUNK)

        def body(j, carry):
            pltpu.async_copy(table_hbm.at[idx_v.at[j]], rows_v, sem).wait()
            pltpu.sync_copy(rows_v, out_hbm.at[pl.ds(base + j * _CHUNK, _CHUNK)])
            return carry

        lax.fori_loop(0, chunks, body, 0)

    return gather_kernel(table2d, idx3d)


def _tc_project(g2d, x_cont, w_full, bias_row, block_b):
    batch, k_g = g2d.shape
    k_c = x_cont.shape[1]
    n_out = w_full.shape[1]

    def body(g_ref, xc_ref, w_ref, b_ref, out_ref):
        g = g_ref[...].astype(jnp.bfloat16)
        x = xc_ref[...].astype(jnp.bfloat16)
        lhs = jnp.concatenate([g, x], axis=1)
        acc = jnp.dot(lhs, w_ref[...], preferred_element_type=jnp.float32)
        out_ref[...] = acc + b_ref[...]

    return pl.pallas_call(
        body,
        grid=(batch // block_b,),
        in_specs=[
            pl.BlockSpec((block_b, k_g), lambda i: (i, 0)),
            pl.BlockSpec((block_b, k_c), lambda i: (i, 0)),
            pl.BlockSpec((k_g + k_c, n_out), lambda i: (0, 0)),
            pl.BlockSpec((1, n_out), lambda i: (0, 0)),
        ],
        out_specs=pl.BlockSpec((block_b, n_out), lambda i: (i, 0)),
        out_shape=jax.ShapeDtypeStruct((batch, n_out), jnp.float32),
    )(g2d, x_cont, w_full, bias_row)


def kernel(x_cat, x_cont, cat_tables, cat_W, cat_b, cont_W, cont_b, cls_token):
    batch, f_cat = x_cat.shape
    f_cont = x_cont.shape[1]
    _, vocab, emb = cat_tables.shape
    d = cat_W.shape[2]

    idx = x_cat.astype(jnp.int32) + (jnp.arange(f_cat, dtype=jnp.int32) * vocab)[None, :]
    n_rows = batch * f_cat
    idx3d = idx.reshape(_NW, n_rows // (_NW * _CHUNK), _CHUNK)
    table2d = cat_tables.reshape(f_cat * vocab, emb)

    gathered = _sc_gather(table2d, idx3d, n_rows, emb)
    g2d = gathered.reshape(batch, f_cat * emb)

    wdt = cat_W.dtype
    eye_cat = jnp.eye(f_cat, dtype=wdt)
    w_cat = (eye_cat[:, None, :, None] * cat_W[:, :, None, :]).reshape(f_cat * emb, f_cat * d)
    eye_cont = jnp.eye(f_cont, dtype=wdt)
    w_cont = (eye_cont[:, :, None] * cont_W[:, None, :]).reshape(f_cont, f_cont * d)
    n_out = (1 + f_cat + f_cont) * d
    top = jnp.concatenate(
        [jnp.zeros((f_cat * emb, d), wdt), w_cat, jnp.zeros((f_cat * emb, f_cont * d), wdt)],
        axis=1)
    bot = jnp.concatenate([jnp.zeros((f_cont, (1 + f_cat) * d), wdt), w_cont], axis=1)
    w_full = jnp.concatenate([top, bot], axis=0).astype(jnp.bfloat16)
    bias_row = jnp.concatenate(
        [cls_token.reshape(d), cat_b.reshape(-1), cont_b.reshape(-1)]).reshape(1, n_out)

    out2d = _tc_project(g2d, x_cont, w_full, bias_row, block_b=256)
    return out2d.reshape(batch, 1 + f_cat + f_cont, d)
